# baseline (speedup 1.0000x reference)
import functools

import jax
import jax.numpy as jnp
import numpy as np
from jax import lax
from jax.experimental import pallas as pl
from jax.experimental.pallas import tpu as pltpu
from jax.experimental.pallas import tpu_sc as plsc

N = 10000
E = 320000
D = 128
H = 128
C = 40

HW = 80
W2EXT = 48

NC = 2
NS = 16
NW = NC * NS
K = 128
NCHUNK = 2560
E_PAD = NCHUNK * K
NSLOT = 4
SB = 20
N_PAD = 10112
ROWS_PER_TILE = N_PAD // NS

BLK = 1000

_PAD_SRC = np.arange(E_PAD - E, dtype=np.int32) % N
_PAD_DST = N + np.arange(E_PAD - E, dtype=np.int32) % (N_PAD - N)


def _make_sc_agg(width, feature_split):
    mesh = plsc.VectorSubcoreMesh(
        core_axis_name="c", subcore_axis_name="s", num_cores=NC, num_subcores=NS
    )
    cpt = (NCHUNK // NS) if feature_split else (NCHUNK // NW)

    @functools.partial(
        pl.kernel,
        mesh=mesh,
        compiler_params=pltpu.CompilerParams(use_tc_tiling_on_sc=False),
        out_type=jax.ShapeDtypeStruct((NC, N_PAD, width), jnp.float32),
        scratch_types=(
            [pltpu.VMEM((SB, K), jnp.int32) for _ in range(2)]
            + [pltpu.VMEM((K, width), jnp.float32) for _ in range(NSLOT)]
            + [pltpu.VMEM_SHARED((N_PAD, width), jnp.float32)]
            + [pltpu.SemaphoreType.DMA for _ in range(2 * NSLOT)]
        ),
    )
    def agg(h_hbm, src_hbm, dst_hbm, zeros_hbm, out_hbm, *scr):
        idx_s = scr[0]
        idx_d = scr[1]
        rows = scr[2:2 + NSLOT]
        acc = scr[2 + NSLOT]
        gsem = scr[3 + NSLOT:3 + 2 * NSLOT]
        ssem = scr[3 + 2 * NSLOT:3 + 3 * NSLOT]
        c = lax.axis_index("c")
        s = lax.axis_index("s")
        r0 = s * ROWS_PER_TILE

        nfull = ROWS_PER_TILE // K
        for f in range(nfull):
            pltpu.sync_copy(zeros_hbm, acc.at[pl.ds(r0 + f * K, K)])
        rem = ROWS_PER_TILE - nfull * K
        if rem:
            pltpu.sync_copy(zeros_hbm.at[pl.ds(0, rem)],
                            acc.at[pl.ds(r0 + nfull * K, rem)])
        plsc.subcore_barrier()

        h_view = h_hbm.at[c] if feature_split else h_hbm
        row0 = (s if feature_split else c * NS + s) * cpt

        @pl.loop(0, cpt // SB)
        def _(b):
            blk = row0 + b * SB
            pltpu.sync_copy(src_hbm.at[pl.ds(blk, SB)], idx_s)
            pltpu.sync_copy(dst_hbm.at[pl.ds(blk, SB)], idx_d)

            for t in range(NSLOT):
                pltpu.async_copy(h_view.at[idx_s.at[t]], rows[t], gsem[t])
            for t in range(NSLOT):
                pltpu.make_async_copy(
                    h_view.at[idx_s.at[t]], rows[t], gsem[t]).wait()
                pltpu.async_copy(rows[t], acc.at[idx_d.at[t]], ssem[t], add=True)

            @pl.loop(NSLOT, SB, step=NSLOT)
            def _(j):
                for t in range(NSLOT):
                    pltpu.make_async_copy(
                        rows[t], acc.at[idx_d.at[j + t - NSLOT]], ssem[t]).wait()
                    pltpu.async_copy(h_view.at[idx_s.at[j + t]], rows[t], gsem[t])
                for t in range(NSLOT):
                    pltpu.make_async_copy(
                        h_view.at[idx_s.at[j + t]], rows[t], gsem[t]).wait()
                    pltpu.async_copy(rows[t], acc.at[idx_d.at[j + t]], ssem[t],
                                     add=True)

            for t in range(NSLOT):
                pltpu.make_async_copy(
                    rows[t], acc.at[idx_d.at[SB + t - NSLOT]], ssem[t]).wait()

        plsc.subcore_barrier()
        pltpu.sync_copy(acc.at[pl.ds(r0, ROWS_PER_TILE)],
                        out_hbm.at[c, pl.ds(r0, ROWS_PER_TILE)])

    return agg


_agg1 = _make_sc_agg(HW, feature_split=True)
_agg2 = _make_sc_agg(W2EXT, feature_split=False)


def _mm1_body(x_ref, w_ref, o_ref):
    h = jnp.dot(x_ref[...], w_ref[...],
                preferred_element_type=jnp.float32,
                precision=lax.Precision.HIGHEST)
    o_ref[0] = h[:, :HW]
    col = lax.broadcasted_iota(jnp.int32, (BLK, 2 * HW - D), 1)
    o_ref[1, :, :D - HW] = h[:, HW:]
    o_ref[1, :, D - HW:] = jnp.where(col == 0, 1.0, 0.0)


def _mm1(x, w1):
    return pl.pallas_call(
        _mm1_body,
        grid=(N // BLK,),
        in_specs=[
            pl.BlockSpec((BLK, D), lambda i: (i, 0)),
            pl.BlockSpec((D, H), lambda i: (0, 0)),
        ],
        out_specs=pl.BlockSpec((NC, BLK, HW), lambda i: (0, i, 0)),
        out_shape=jax.ShapeDtypeStruct((NC, N, HW), jnp.float32),
    )(x, w1)


def _fin1_body(a_ref, b1_ref, w2_ref, h2_ref, rdeg_ref):
    half0 = a_ref[0]
    half1 = a_ref[1]
    su = jnp.concatenate([half0, half1[:, :D - HW]], axis=1)
    deg = jnp.maximum(half1[:, D - HW:D - HW + 1], 1.0)
    rdeg = 1.0 / deg
    h1 = jnp.maximum(su * rdeg + b1_ref[...], 0.0)
    h2_ref[...] = jnp.dot(h1, w2_ref[...],
                          preferred_element_type=jnp.float32,
                          precision=lax.Precision.HIGHEST)
    rdeg_ref[...] = rdeg


def _fin1(acc, b1, w2p):
    return pl.pallas_call(
        _fin1_body,
        grid=(N // BLK,),
        in_specs=[
            pl.BlockSpec((NC, BLK, HW), lambda i: (0, i, 0)),
            pl.BlockSpec((1, H), lambda i: (0, 0)),
            pl.BlockSpec((H, W2EXT), lambda i: (0, 0)),
        ],
        out_specs=[
            pl.BlockSpec((BLK, W2EXT), lambda i: (i, 0)),
            pl.BlockSpec((BLK, 1), lambda i: (i, 0)),
        ],
        out_shape=[
            jax.ShapeDtypeStruct((N, W2EXT), jnp.float32),
            jax.ShapeDtypeStruct((N, 1), jnp.float32),
        ],
    )(acc, b1, w2p)


def _fin2_body(a_ref, rdeg_ref, b2_ref, o_ref):
    o = (a_ref[0] + a_ref[1]) * rdeg_ref[...] + b2_ref[...]
    o_ref[...] = o[:, :C]


def _fin2(acc, rdeg, b2p):
    return pl.pallas_call(
        _fin2_body,
        grid=(N // BLK,),
        in_specs=[
            pl.BlockSpec((NC, BLK, W2EXT), lambda i: (0, i, 0)),
            pl.BlockSpec((BLK, 1), lambda i: (i, 0)),
            pl.BlockSpec((1, W2EXT), lambda i: (0, 0)),
        ],
        out_specs=pl.BlockSpec((BLK, C), lambda i: (i, 0)),
        out_shape=jax.ShapeDtypeStruct((N, C), jnp.float32),
    )(acc, rdeg, b2p)


def kernel(x, edge_index, W1, b1, W2, b2):
    src = jnp.concatenate(
        [edge_index[0], jnp.asarray(_PAD_SRC)]).reshape(NCHUNK, K)
    dst = jnp.concatenate(
        [edge_index[1], jnp.asarray(_PAD_DST)]).reshape(NCHUNK, K)

    hs = _mm1(x, W1)
    zeros1 = jnp.zeros((K, HW), jnp.float32)
    acc1 = _agg1(hs, src, dst, zeros1)

    w2p = jnp.pad(W2, ((0, 0), (0, W2EXT - C)))
    h2, rdeg = _fin1(acc1, b1.reshape(1, H), w2p)

    zeros2 = jnp.zeros((K, W2EXT), jnp.float32)
    acc2 = _agg2(h2, src, dst, zeros2)

    b2p = jnp.pad(b2, (0, W2EXT - C)).reshape(1, W2EXT)
    return _fin2(acc2, rdeg, b2p)


# speedup vs baseline: 1.1267x; 1.1267x over previous
import functools

import jax
import jax.numpy as jnp
import numpy as np
from jax import lax
from jax.experimental import pallas as pl
from jax.experimental.pallas import tpu as pltpu
from jax.experimental.pallas import tpu_sc as plsc

N = 10000
E = 320000
D = 128
H = 128
C = 40

HW = 72
W2EXT = 48

NC = 2
NS = 16
NW = NC * NS
K = 128
NCHUNK = 2560
E_PAD = NCHUNK * K
NSLOT = 4
SB = 20
N_PAD = 10112
ROWS_PER_TILE = N_PAD // NS

BLK = 2000

_PAD_SRC = np.arange(E_PAD - E, dtype=np.int32) % N
_PAD_DST = N + np.arange(E_PAD - E, dtype=np.int32) % (N_PAD - N)


def _make_sc_agg(width, feature_split):
    mesh = plsc.VectorSubcoreMesh(
        core_axis_name="c", subcore_axis_name="s", num_cores=NC, num_subcores=NS
    )
    cpt = (NCHUNK // NS) if feature_split else (NCHUNK // NW)

    @functools.partial(
        pl.kernel,
        mesh=mesh,
        compiler_params=pltpu.CompilerParams(use_tc_tiling_on_sc=False),
        out_type=jax.ShapeDtypeStruct((NC, N_PAD, width), jnp.float32),
        scratch_types=(
            [pltpu.VMEM((SB, K), jnp.int32) for _ in range(2)]
            + [pltpu.VMEM((K, width), jnp.float32) for _ in range(NSLOT)]
            + [pltpu.VMEM_SHARED((N_PAD, width), jnp.float32)]
            + [pltpu.SemaphoreType.DMA for _ in range(2 * NSLOT)]
        ),
    )
    def agg(h_hbm, src_hbm, dst_hbm, zeros_hbm, out_hbm, *scr):
        idx_s = scr[0]
        idx_d = scr[1]
        rows = scr[2:2 + NSLOT]
        acc = scr[2 + NSLOT]
        gsem = scr[3 + NSLOT:3 + 2 * NSLOT]
        ssem = scr[3 + 2 * NSLOT:3 + 3 * NSLOT]
        c = lax.axis_index("c")
        s = lax.axis_index("s")
        r0 = s * ROWS_PER_TILE

        pltpu.sync_copy(zeros_hbm.at[pl.ds(r0, ROWS_PER_TILE)],
                        acc.at[pl.ds(r0, ROWS_PER_TILE)])
        plsc.subcore_barrier()

        h_view = h_hbm.at[c] if feature_split else h_hbm
        row0 = (s if feature_split else c * NS + s) * cpt

        @pl.loop(0, cpt // SB)
        def _(b):
            blk = row0 + b * SB
            pltpu.sync_copy(src_hbm.at[pl.ds(blk, SB)], idx_s)
            pltpu.sync_copy(dst_hbm.at[pl.ds(blk, SB)], idx_d)

            for t in range(NSLOT):
                pltpu.async_copy(h_view.at[idx_s.at[t]], rows[t], gsem[t])
            for t in range(NSLOT):
                pltpu.make_async_copy(
                    h_view.at[idx_s.at[t]], rows[t], gsem[t]).wait()
                pltpu.async_copy(rows[t], acc.at[idx_d.at[t]], ssem[t], add=True)

            @pl.loop(NSLOT, SB, step=NSLOT)
            def _(j):
                for t in range(NSLOT):
                    pltpu.make_async_copy(
                        rows[t], acc.at[idx_d.at[j + t - NSLOT]], ssem[t]).wait()
                    pltpu.async_copy(h_view.at[idx_s.at[j + t]], rows[t], gsem[t])
                for t in range(NSLOT):
                    pltpu.make_async_copy(
                        h_view.at[idx_s.at[j + t]], rows[t], gsem[t]).wait()
                    pltpu.async_copy(rows[t], acc.at[idx_d.at[j + t]], ssem[t],
                                     add=True)

            for t in range(NSLOT):
                pltpu.make_async_copy(
                    rows[t], acc.at[idx_d.at[SB + t - NSLOT]], ssem[t]).wait()

        plsc.subcore_barrier()
        pltpu.sync_copy(acc.at[pl.ds(r0, ROWS_PER_TILE)],
                        out_hbm.at[c, pl.ds(r0, ROWS_PER_TILE)])

    return agg


_agg1 = _make_sc_agg(HW, feature_split=True)
_agg2 = _make_sc_agg(W2EXT, feature_split=False)


def _mm1_body(x_ref, w_ref, o_ref):
    h = jnp.dot(x_ref[...], w_ref[...],
                preferred_element_type=jnp.float32,
                precision=lax.Precision.HIGHEST)
    o_ref[0] = h[:, :HW]
    col = lax.broadcasted_iota(jnp.int32, (BLK, 2 * HW - D), 1)
    o_ref[1, :, :D - HW] = h[:, HW:]
    o_ref[1, :, D - HW:] = jnp.where(col == 0, 1.0, 0.0)


def _mm1(x, w1):
    return pl.pallas_call(
        _mm1_body,
        grid=(N // BLK,),
        in_specs=[
            pl.BlockSpec((BLK, D), lambda i: (i, 0)),
            pl.BlockSpec((D, H), lambda i: (0, 0)),
        ],
        out_specs=pl.BlockSpec((NC, BLK, HW), lambda i: (0, i, 0)),
        out_shape=jax.ShapeDtypeStruct((NC, N, HW), jnp.float32),
    )(x, w1)


def _fin1_body(a_ref, b1_ref, w2_ref, h2_ref, rdeg_ref):
    half0 = a_ref[0]
    half1 = a_ref[1]
    su = jnp.concatenate([half0, half1[:, :D - HW]], axis=1)
    deg = jnp.maximum(half1[:, D - HW:D - HW + 1], 1.0)
    rdeg = 1.0 / deg
    h1 = jnp.maximum(su * rdeg + b1_ref[...], 0.0)
    h2_ref[...] = jnp.dot(h1, w2_ref[...],
                          preferred_element_type=jnp.float32,
                          precision=lax.Precision.HIGHEST)
    rdeg_ref[...] = rdeg


def _fin1(acc, b1, w2p):
    return pl.pallas_call(
        _fin1_body,
        grid=(N // BLK,),
        in_specs=[
            pl.BlockSpec((NC, BLK, HW), lambda i: (0, i, 0)),
            pl.BlockSpec((1, H), lambda i: (0, 0)),
            pl.BlockSpec((H, W2EXT), lambda i: (0, 0)),
        ],
        out_specs=[
            pl.BlockSpec((BLK, W2EXT), lambda i: (i, 0)),
            pl.BlockSpec((BLK, 1), lambda i: (i, 0)),
        ],
        out_shape=[
            jax.ShapeDtypeStruct((N, W2EXT), jnp.float32),
            jax.ShapeDtypeStruct((N, 1), jnp.float32),
        ],
    )(acc, b1, w2p)


def _fin2_body(a_ref, rdeg_ref, b2_ref, o_ref):
    o = (a_ref[0] + a_ref[1]) * rdeg_ref[...] + b2_ref[...]
    o_ref[...] = o[:, :C]


def _fin2(acc, rdeg, b2p):
    return pl.pallas_call(
        _fin2_body,
        grid=(N // BLK,),
        in_specs=[
            pl.BlockSpec((NC, BLK, W2EXT), lambda i: (0, i, 0)),
            pl.BlockSpec((BLK, 1), lambda i: (i, 0)),
            pl.BlockSpec((1, W2EXT), lambda i: (0, 0)),
        ],
        out_specs=pl.BlockSpec((BLK, C), lambda i: (i, 0)),
        out_shape=jax.ShapeDtypeStruct((N, C), jnp.float32),
    )(acc, rdeg, b2p)


def kernel(x, edge_index, W1, b1, W2, b2):
    src = jnp.concatenate(
        [edge_index[0], jnp.asarray(_PAD_SRC)]).reshape(NCHUNK, K)
    dst = jnp.concatenate(
        [edge_index[1], jnp.asarray(_PAD_DST)]).reshape(NCHUNK, K)

    hs = _mm1(x, W1)
    zeros1 = jnp.zeros((N_PAD, HW), jnp.float32)
    acc1 = _agg1(hs, src, dst, zeros1)

    w2p = jnp.pad(W2, ((0, 0), (0, W2EXT - C)))
    h2, rdeg = _fin1(acc1, b1.reshape(1, H), w2p)

    zeros2 = jnp.zeros((N_PAD, W2EXT), jnp.float32)
    acc2 = _agg2(h2, src, dst, zeros2)

    b2p = jnp.pad(b2, (0, W2EXT - C)).reshape(1, W2EXT)
    return _fin2(acc2, rdeg, b2p)
